# baseline (speedup 1.0000x reference)
import functools

import jax
import jax.numpy as jnp
from jax import lax
from jax.experimental import pallas as pl
from jax.experimental.pallas import tpu as pltpu
from jax.experimental.pallas import tpu_sc as plsc

SIZE = 256
HALF_EXTENT = 3.0
N_POINTS = 4194304

NC = 2
NS = 16
L = 16
NW = NC * NS
C = 2048
PER_W = N_POINTS // NW
N_CH = PER_W // C

KS = SIZE / (2.0 * HALF_EXTENT)
KO = (SIZE - 1.0) / 2.0


def _perm(v, idx):
    return lax.gather(
        v, idx[:, None],
        dimension_numbers=lax.GatherDimensionNumbers(
            offset_dims=(), collapsed_slice_dims=(0,), start_index_map=(0,)),
        slice_sizes=(1,),
        mode=lax.GatherScatterMode.PROMISE_IN_BOUNDS)


def _sc_grid_sample(x_flat, data_flat):
    mesh = plsc.VectorSubcoreMesh(core_axis_name="c", subcore_axis_name="s")

    scratch = [pltpu.VMEM((3 * C,), jnp.float32) for _ in range(2)]
    scratch += [pltpu.VMEM((8 * C,), jnp.int32) for _ in range(2)]
    scratch += [pltpu.VMEM((8 * C,), jnp.float32) for _ in range(2)]
    scratch += [pltpu.VMEM((8 * C,), jnp.float32) for _ in range(2)]
    scratch += [pltpu.VMEM((C,), jnp.float32)]
    scratch += [pltpu.SemaphoreType.DMA for _ in range(4)]

    @functools.partial(
        pl.kernel,
        mesh=mesh,
        out_type=jax.ShapeDtypeStruct((N_POINTS,), jnp.float32),
        scratch_types=scratch,
    )
    def k(x_hbm, data_hbm, out_hbm, *rest):
        coord_ab = rest[0:2]
        idx_ab = rest[2:4]
        val_ab = rest[4:6]
        w_ab = rest[6:8]
        out_b = rest[8]
        gsem = rest[9:11]
        csem = rest[11:13]

        wid = lax.axis_index("s") * NC + lax.axis_index("c")
        lane = lax.iota(jnp.int32, L)
        lo_a = (5, 4, 4)
        lo_b = (10, 10, 9)

        def fire_coords(i, p):
            base = wid * PER_W + i * C
            pltpu.async_copy(
                x_hbm.at[pl.ds(3 * base, 3 * C)], coord_ab[p], csem[p])

        def drain_coords(i, p):
            base = wid * PER_W + i * C
            pltpu.make_async_copy(
                x_hbm.at[pl.ds(3 * base, 3 * C)], coord_ab[p], csem[p]).wait()

        def compute(p):
            coord_b = coord_ab[p]
            idx_b = idx_ab[p]
            w_b = w_ab[p]

            def jbody(j, carry):
                pb = j * L
                r0 = coord_b[pl.ds(3 * pb, L)]
                r1 = coord_b[pl.ds(3 * pb + L, L)]
                r2 = coord_b[pl.ds(3 * pb + 2 * L, L)]
                axes = []
                for comp in range(3):
                    src = (3 * lane + comp) & (L - 1)
                    cc = jnp.where(
                        lane <= lo_a[comp], _perm(r0, src),
                        jnp.where(lane <= lo_b[comp], _perm(r1, src),
                                  _perm(r2, src)))
                    t = cc * KS + KO
                    ti = t.astype(jnp.int32)
                    i0 = jnp.where(ti.astype(jnp.float32) > t, ti - 1, ti)
                    f = t - i0.astype(jnp.float32)
                    v0 = (i0 >= 0) & (i0 <= SIZE - 1)
                    v1 = (i0 >= -1) & (i0 <= SIZE - 2)
                    c0 = jnp.minimum(jnp.maximum(i0, 0), SIZE - 1)
                    c1 = jnp.minimum(jnp.maximum(i0 + 1, 0), SIZE - 1)
                    w0 = jnp.where(v0, 1.0 - f, 0.0)
                    w1 = jnp.where(v1, f, 0.0)
                    axes.append((c0, c1, w0, w1))
                (cx0, cx1, wx0, wx1) = axes[0]
                (cy0, cy1, wy0, wy1) = axes[1]
                (cz0, cz1, wz0, wz1) = axes[2]
                for dz in (0, 1):
                    zb = (cz1 if dz else cz0) * (SIZE * SIZE)
                    wz = wz1 if dz else wz0
                    for dy in (0, 1):
                        rb = zb + (cy1 if dy else cy0) * SIZE
                        wzy = wz * (wy1 if dy else wy0)
                        for dx in (0, 1):
                            kk = dz * 4 + dy * 2 + dx
                            idx_b[pl.ds(kk * C + pb, L)] = rb + (cx1 if dx else cx0)
                            w_b[pl.ds(kk * C + pb, L)] = wzy * (wx1 if dx else wx0)
                return carry

            lax.fori_loop(0, C // L, jbody, 0)

        def fire(p):
            pltpu.async_copy(data_hbm.at[idx_ab[p]], val_ab[p], gsem[p])

        def drain(p):
            pltpu.make_async_copy(data_hbm.at[idx_ab[p]], val_ab[p], gsem[p]).wait()

        def combine_store(i, p):
            val_b, w_b = val_ab[p], w_ab[p]

            def jbody(j, carry):
                pb = j * L
                acc = w_b[pl.ds(pb, L)] * val_b[pl.ds(pb, L)]
                for kk in range(1, 8):
                    s = kk * C + pb
                    acc = acc + w_b[pl.ds(s, L)] * val_b[pl.ds(s, L)]
                out_b[pl.ds(pb, L)] = acc
                return carry

            lax.fori_loop(0, C // L, jbody, 0)
            base = wid * PER_W + i * C
            pltpu.sync_copy(out_b, out_hbm.at[pl.ds(base, C)])

        def half(i, p):
            drain_coords(i, p)
            fire_coords(i + 1, 1 - p)
            compute(p)
            drain(1 - p)
            fire(p)
            combine_store(i - 1, 1 - p)

        fire_coords(0, 0)
        drain_coords(0, 0)
        fire_coords(1, 1)
        compute(0)
        fire(0)

        def body(s, carry):
            i = 2 * s + 1
            half(i, 1)
            half(i + 1, 0)
            return carry

        lax.fori_loop(0, (N_CH - 2) // 2, body, 0)

        drain_coords(N_CH - 1, 1)
        compute(1)
        drain(0)
        fire(1)
        combine_store(N_CH - 2, 0)
        drain(1)
        combine_store(N_CH - 1, 1)

    return k(x_flat, data_flat)


def kernel(x, data):
    x_shape = x.shape
    xf = x.reshape(-1)
    df = data.reshape(-1)
    out = _sc_grid_sample(xf, df)
    return out.reshape(x_shape[:-1])


# speedup vs baseline: 4.8699x; 4.8699x over previous
import functools

import jax
import jax.numpy as jnp
from jax import lax
from jax.experimental import pallas as pl
from jax.experimental.pallas import tpu as pltpu
from jax.experimental.pallas import tpu_sc as plsc

SIZE = 256
HALF_EXTENT = 3.0
N_POINTS = 4194304

NC = 2
NS = 16
L = 16
NW = NC * NS
C = 2048
PER_W = N_POINTS // NW
N_CH = PER_W // C

KS = SIZE / (2.0 * HALF_EXTENT)
KO = (SIZE - 1.0) / 2.0


def _sc_grid_sample(cx, cy, cz, data_flat):
    mesh = plsc.VectorSubcoreMesh(core_axis_name="c", subcore_axis_name="s")

    scratch = [pltpu.VMEM((C,), jnp.float32) for _ in range(6)]
    scratch += [pltpu.VMEM((8 * C,), jnp.int32) for _ in range(2)]
    scratch += [pltpu.VMEM((8 * C,), jnp.float32) for _ in range(2)]
    scratch += [pltpu.VMEM((8 * C,), jnp.float32) for _ in range(2)]
    scratch += [pltpu.VMEM((C,), jnp.float32)]
    scratch += [pltpu.SemaphoreType.DMA for _ in range(4)]

    @functools.partial(
        pl.kernel,
        mesh=mesh,
        out_type=jax.ShapeDtypeStruct((N_POINTS,), jnp.float32),
        scratch_types=scratch,
    )
    def k(cx_hbm, cy_hbm, cz_hbm, data_hbm, out_hbm, *rest):
        c_hbm = (cx_hbm, cy_hbm, cz_hbm)
        coord_ab = (rest[0:3], rest[3:6])
        idx_ab = rest[6:8]
        val_ab = rest[8:10]
        w_ab = rest[10:12]
        out_b = rest[12]
        gsem = rest[13:15]
        csem = rest[15:17]

        wid = lax.axis_index("s") * NC + lax.axis_index("c")

        def fire_coords(i, p):
            base = wid * PER_W + i * C
            for comp in range(3):
                pltpu.async_copy(
                    c_hbm[comp].at[pl.ds(base, C)], coord_ab[p][comp], csem[p])

        def drain_coords(i, p):
            base = wid * PER_W + i * C
            for comp in range(3):
                pltpu.make_async_copy(
                    c_hbm[comp].at[pl.ds(base, C)], coord_ab[p][comp],
                    csem[p]).wait()

        def compute(p):
            coord_b = coord_ab[p]
            idx_b = idx_ab[p]
            w_b = w_ab[p]

            def jbody(j, carry):
                pb = j * L
                axes = []
                for comp in range(3):
                    cc = coord_b[comp][pl.ds(pb, L)]
                    t = cc * KS + KO
                    ti = t.astype(jnp.int32)
                    i0 = jnp.where(ti.astype(jnp.float32) > t, ti - 1, ti)
                    f = t - i0.astype(jnp.float32)
                    v0 = (i0 >= 0) & (i0 <= SIZE - 1)
                    v1 = (i0 >= -1) & (i0 <= SIZE - 2)
                    c0 = jnp.minimum(jnp.maximum(i0, 0), SIZE - 1)
                    c1 = jnp.minimum(jnp.maximum(i0 + 1, 0), SIZE - 1)
                    w0 = jnp.where(v0, 1.0 - f, 0.0)
                    w1 = jnp.where(v1, f, 0.0)
                    axes.append((c0, c1, w0, w1))
                (cx0, cx1, wx0, wx1) = axes[0]
                (cy0, cy1, wy0, wy1) = axes[1]
                (cz0, cz1, wz0, wz1) = axes[2]
                for dz in (0, 1):
                    zb = (cz1 if dz else cz0) * (SIZE * SIZE)
                    wz = wz1 if dz else wz0
                    for dy in (0, 1):
                        rb = zb + (cy1 if dy else cy0) * SIZE
                        wzy = wz * (wy1 if dy else wy0)
                        for dx in (0, 1):
                            kk = dz * 4 + dy * 2 + dx
                            idx_b[pl.ds(kk * C + pb, L)] = rb + (cx1 if dx else cx0)
                            w_b[pl.ds(kk * C + pb, L)] = wzy * (wx1 if dx else wx0)
                return carry

            lax.fori_loop(0, C // L, jbody, 0)

        def fire(p):
            pltpu.async_copy(data_hbm.at[idx_ab[p]], val_ab[p], gsem[p])

        def drain(p):
            pltpu.make_async_copy(data_hbm.at[idx_ab[p]], val_ab[p], gsem[p]).wait()

        def combine_store(i, p):
            val_b, w_b = val_ab[p], w_ab[p]

            def jbody(j, carry):
                pb = j * L
                acc = w_b[pl.ds(pb, L)] * val_b[pl.ds(pb, L)]
                for kk in range(1, 8):
                    s = kk * C + pb
                    acc = acc + w_b[pl.ds(s, L)] * val_b[pl.ds(s, L)]
                out_b[pl.ds(pb, L)] = acc
                return carry

            lax.fori_loop(0, C // L, jbody, 0)
            base = wid * PER_W + i * C
            pltpu.sync_copy(out_b, out_hbm.at[pl.ds(base, C)])

        def half(i, p):
            drain_coords(i, p)
            fire_coords(i + 1, 1 - p)
            compute(p)
            drain(1 - p)
            fire(p)
            combine_store(i - 1, 1 - p)

        fire_coords(0, 0)
        drain_coords(0, 0)
        fire_coords(1, 1)
        compute(0)
        fire(0)

        def body(s, carry):
            i = 2 * s + 1
            half(i, 1)
            half(i + 1, 0)
            return carry

        lax.fori_loop(0, (N_CH - 2) // 2, body, 0)

        drain_coords(N_CH - 1, 1)
        compute(1)
        drain(0)
        fire(1)
        combine_store(N_CH - 2, 0)
        drain(1)
        combine_store(N_CH - 1, 1)

    return k(cx, cy, cz, data_flat)


def kernel(x, data):
    x_shape = x.shape
    xf = x.reshape(-1, 3)
    df = data.reshape(-1)
    out = _sc_grid_sample(xf[:, 0], xf[:, 1], xf[:, 2], df)
    return out.reshape(x_shape[:-1])
